# baseline (speedup 1.0000x reference)
import jax
import jax.numpy as jnp
from jax.experimental import pallas as pl
from jax.experimental.pallas import tpu as pltpu

NUM_EXPERTS = 8
TOP_K = 2
CHALF = 512


def _moe_kernel(x_ref, rw_ref, w1_ref, w2_ref, out_ref, hA, hB, gates_ref):
    e = pl.program_id(0)
    p = pl.program_id(1)

    @pl.when((e == 0) & (p == 0))
    def _():
        x = x_ref[...]
        rw = rw_ref[...]
        logits = jax.lax.dot_general(
            x, rw, (((1,), (1,)), ((), ())),
            preferred_element_type=jnp.float32)
        m = jnp.max(logits, axis=-1, keepdims=True)
        el = jnp.exp(logits - m)
        probs = el / jnp.sum(el, axis=-1, keepdims=True)
        col = jax.lax.broadcasted_iota(jnp.int32, probs.shape, 1)
        big = jnp.int32(NUM_EXPERTS)
        m1 = jnp.max(probs, axis=-1, keepdims=True)
        is1 = probs == m1
        idx1 = jnp.min(jnp.where(is1, col, big), axis=-1, keepdims=True)
        first1 = col == idx1
        probs_wo1 = jnp.where(first1, -1.0, probs)
        m2 = jnp.max(probs_wo1, axis=-1, keepdims=True)
        is2 = probs_wo1 == m2
        idx2 = jnp.min(jnp.where(is2, col, big), axis=-1, keepdims=True)
        first2 = col == idx2
        gates_ref[...] = jnp.where(first1 | first2, probs, 0.0)

    @pl.when(e < NUM_EXPERTS)
    def _():
        xc = x_ref[:, pl.ds(p * CHALF, CHALF)]
        hw = jnp.dot(xc, w1_ref[0], preferred_element_type=jnp.float32)

        even = jax.lax.rem(e, 2) == 0

        @pl.when(even & (p == 0))
        def _():
            hA[...] = hw

        @pl.when(even & (p == 1))
        def _():
            hA[...] += hw

        @pl.when((~even) & (p == 0))
        def _():
            hB[...] = hw

        @pl.when((~even) & (p == 1))
        def _():
            hB[...] += hw

    @pl.when(e > 0)
    def _():
        ec = e - 1
        even_c = jax.lax.rem(ec, 2) == 0
        ha = hA[:, pl.ds(p * CHALF, CHALF)]
        hb = hB[:, pl.ds(p * CHALF, CHALF)]
        h = jnp.where(even_c, ha, hb)
        h = jnp.maximum(h, 0.0)
        y = jnp.dot(h, w2_ref[0], preferred_element_type=jnp.float32)

        gates = gates_ref[...]
        col = jax.lax.broadcasted_iota(jnp.int32, gates.shape, 1)
        g = jnp.sum(jnp.where(col == ec, gates, 0.0), axis=-1,
                    keepdims=True)
        contrib = g * y

        @pl.when((e == 1) & (p == 0))
        def _():
            out_ref[...] = contrib

        @pl.when((e != 1) | (p != 0))
        def _():
            out_ref[...] += contrib


def kernel(x, router_w, w1, w2):
    B, T, C = x.shape
    N = B * T
    E, _, F = w1.shape
    x_flat = x.reshape(N, C)
    EM1 = E - 1

    out = pl.pallas_call(
        _moe_kernel,
        grid=(E + 1, 2),
        in_specs=[
            pl.BlockSpec((N, C), lambda e, p: (0, 0)),
            pl.BlockSpec((E, C), lambda e, p: (0, 0)),
            pl.BlockSpec((1, CHALF, F), lambda e, p: (jnp.minimum(e, EM1), p, 0)),
            pl.BlockSpec((1, CHALF, C),
                         lambda e, p: (jnp.maximum(e - 1, 0), p, 0)),
        ],
        out_specs=pl.BlockSpec((N, C), lambda e, p: (0, 0)),
        out_shape=jax.ShapeDtypeStruct((N, C), jnp.float32),
        scratch_shapes=[
            pltpu.VMEM((N, F), jnp.float32),
            pltpu.VMEM((N, F), jnp.float32),
            pltpu.VMEM((N, NUM_EXPERTS), jnp.float32),
        ],
    )(x_flat, router_w, w1, w2)
    return out.reshape(B, T, C)


# speedup vs baseline: 1.2347x; 1.2347x over previous
import jax
import jax.numpy as jnp
from jax.experimental import pallas as pl

NUM_EXPERTS = 8
TOP_K = 2
EPB = 2


def _moe_kernel(x_ref, rw_ref, w1_ref, w2_ref, out_ref):
    i = pl.program_id(0)
    x = x_ref[...]
    rw = rw_ref[...]

    logits = jax.lax.dot_general(
        x, rw, (((1,), (1,)), ((), ())),
        preferred_element_type=jnp.float32)
    m = jnp.max(logits, axis=-1, keepdims=True)
    el = jnp.exp(logits - m)
    probs = el / jnp.sum(el, axis=-1, keepdims=True)

    col = jax.lax.broadcasted_iota(jnp.int32, probs.shape, 1)
    big = jnp.int32(NUM_EXPERTS)
    m1 = jnp.max(probs, axis=-1, keepdims=True)
    is1 = probs == m1
    idx1 = jnp.min(jnp.where(is1, col, big), axis=-1, keepdims=True)
    first1 = col == idx1
    probs_wo1 = jnp.where(first1, -1.0, probs)
    m2 = jnp.max(probs_wo1, axis=-1, keepdims=True)
    is2 = probs_wo1 == m2
    idx2 = jnp.min(jnp.where(is2, col, big), axis=-1, keepdims=True)
    first2 = col == idx2
    gates = jnp.where(first1 | first2, probs, 0.0)

    contrib = None
    for j in range(EPB):
        e = i * EPB + j
        gate_e = jnp.sum(jnp.where(col == e, gates, 0.0), axis=-1,
                         keepdims=True)
        h = jnp.dot(x, w1_ref[j], preferred_element_type=jnp.float32)
        h = jnp.maximum(h, 0.0)
        y = jnp.dot(h, w2_ref[j], preferred_element_type=jnp.float32)
        c = gate_e * y
        contrib = c if contrib is None else contrib + c

    @pl.when(i == 0)
    def _():
        out_ref[...] = contrib

    @pl.when(i != 0)
    def _():
        out_ref[...] += contrib


def kernel(x, router_w, w1, w2):
    B, T, C = x.shape
    N = B * T
    E, _, F = w1.shape
    x_flat = x.reshape(N, C)

    out = pl.pallas_call(
        _moe_kernel,
        grid=(E // EPB,),
        in_specs=[
            pl.BlockSpec((N, C), lambda i: (0, 0)),
            pl.BlockSpec((E, C), lambda i: (0, 0)),
            pl.BlockSpec((EPB, C, F), lambda i: (i, 0, 0)),
            pl.BlockSpec((EPB, F, C), lambda i: (i, 0, 0)),
        ],
        out_specs=pl.BlockSpec((N, C), lambda i: (0, 0)),
        out_shape=jax.ShapeDtypeStruct((N, C), jnp.float32),
    )(x_flat, router_w, w1, w2)
    return out.reshape(B, T, C)
